# baseline (speedup 1.0000x reference)
import functools

import jax
import jax.numpy as jnp
from jax import lax
from jax.experimental import pallas as pl
from jax.experimental.pallas import tpu as pltpu
from jax.experimental.pallas import tpu_sc as plsc

N_TOK = 2048
D = 768
E = 8
H = 2048
K = 2
A_TOT = N_TOK * K
BM = 256
MAXB = A_TOT // BM + E
MAXR = MAXB * BM
HB = 512
NH = H // HB

NW = 32
RPW = MAXR // NW
TPW = N_TOK // NW
GCH = 96

_SC_MESH = dict(core_axis_name="c", subcore_axis_name="s", num_cores=2,
                num_subcores=16)


def _route_body(x_ref, wg_ref, pos0_ref, pos1_ref, w0_ref, w1_ref, meta_ref,
                cum_ref):
    xl = x_ref[...]
    wg = wg_ref[...]
    logits = lax.dot_general(xl, wg, (((1,), (1,)), ((), ())),
                             preferred_element_type=jnp.float32)
    e_io = lax.broadcasted_iota(jnp.int32, (N_TOK, E), 1)
    m0 = jnp.max(logits, axis=1, keepdims=True)
    i0 = jnp.min(jnp.where(logits == m0, e_io, E), axis=1, keepdims=True)
    masked = jnp.where(e_io == i0, -jnp.inf, logits)
    m1 = jnp.max(masked, axis=1, keepdims=True)
    i1 = jnp.min(jnp.where(masked == m1, e_io, E), axis=1, keepdims=True)
    a = jnp.exp(m1 - m0)
    w0_ref[...] = 1.0 / (1.0 + a)
    w1_ref[...] = a / (1.0 + a)

    oh0 = (e_io == i0).astype(jnp.float32)
    oh1 = (e_io == i1).astype(jnp.float32)
    oh = jnp.concatenate([oh0, oh1], axis=1)
    tri = (lax.broadcasted_iota(jnp.int32, (128, 128), 0)
           >= lax.broadcasted_iota(jnp.int32, (128, 128), 1)
           ).astype(jnp.float32)

    cum_ref[...] = oh

    def chunk(c, run):
        blk = cum_ref[pl.ds(c * 128, 128), :]
        p = lax.dot_general(tri, blk, (((1,), (0,)), ((), ())),
                            preferred_element_type=jnp.float32) + run
        cum_ref[pl.ds(c * 128, 128), :] = p
        return p[127:128, :]

    run = lax.fori_loop(0, N_TOK // 128, chunk, jnp.zeros((1, 2 * E),
                                                          jnp.float32))
    cum = cum_ref[...]
    cum0, cum1 = cum[:, 0:E], cum[:, E:2 * E]
    counts0, counts1 = run[:, 0:E], run[:, E:2 * E]
    counts = counts0 + counts1
    nblk = jnp.floor((counts + (BM - 1)) * (1.0 / BM))
    ut = (lax.broadcasted_iota(jnp.int32, (E, E), 0)
          <= lax.broadcasted_iota(jnp.int32, (E, E), 1)).astype(jnp.float32)
    cumblk = lax.dot_general(nblk, ut, (((1,), (0,)), ((), ())),
                             preferred_element_type=jnp.float32)
    pad_off = (cumblk - nblk) * BM
    used = cumblk[:, E - 1:E]

    base0 = pad_off + cum0 - 1.0
    base1 = pad_off + counts0 + cum1 - 1.0
    pos0_ref[...] = jnp.sum(oh0 * base0, axis=1, keepdims=True).astype(
        jnp.int32)
    pos1_ref[...] = jnp.sum(oh1 * base1, axis=1, keepdims=True).astype(
        jnp.int32)

    eye8 = (lax.broadcasted_iota(jnp.int32, (E, E), 0)
            == lax.broadcasted_iota(jnp.int32, (E, E), 1)).astype(jnp.float32)
    cumblk_col = lax.dot_general(eye8, cumblk, (((1,), (1,)), ((), ())),
                                 preferred_element_type=jnp.float32)
    counts_col = lax.dot_general(eye8, counts, (((1,), (1,)), ((), ())),
                                 preferred_element_type=jnp.float32)
    b_io = lax.broadcasted_iota(jnp.int32, (E, 32), 1).astype(jnp.float32)
    ebm = jnp.sum((b_io >= cumblk_col).astype(jnp.float32), axis=0,
                  keepdims=True)
    e_col = lax.broadcasted_iota(jnp.int32, (E, 1), 0).astype(jnp.float32)
    last_e = jnp.max(jnp.where(counts_col > 0.0, e_col, 0.0), axis=0,
                     keepdims=True)
    b_row = lax.broadcasted_iota(jnp.int32, (1, 32), 1).astype(jnp.float32)
    emap = jnp.where(b_row >= used, last_e, ebm)
    meta_ref[...] = jnp.concatenate(
        [emap, jnp.broadcast_to(used, (1, 32))], axis=1).astype(jnp.int32)


def _route(x2d, Wg):
    return pl.pallas_call(
        _route_body,
        out_shape=[
            jax.ShapeDtypeStruct((N_TOK, 1), jnp.int32),
            jax.ShapeDtypeStruct((N_TOK, 1), jnp.int32),
            jax.ShapeDtypeStruct((N_TOK, 1), jnp.float32),
            jax.ShapeDtypeStruct((N_TOK, 1), jnp.float32),
            jax.ShapeDtypeStruct((1, 64), jnp.int32),
        ],
        scratch_shapes=[pltpu.VMEM((N_TOK, 2 * E), jnp.float32)],
    )(x2d, Wg)


@functools.cache
def _make_dispatch():
  kern = functools.partial(
    pl.kernel,
    mesh=plsc.VectorSubcoreMesh(**_SC_MESH),
    out_type=[
        jax.ShapeDtypeStruct((MAXR, D), jnp.float32),
        jax.ShapeDtypeStruct((MAXR,), jnp.float32),
    ],
    scratch_types=[
        pltpu.VMEM((N_TOK,), jnp.int32),
        pltpu.VMEM((N_TOK,), jnp.int32),
        pltpu.VMEM((N_TOK,), jnp.float32),
        pltpu.VMEM((N_TOK,), jnp.float32),
        pltpu.VMEM((MAXR,), jnp.int32),
        pltpu.VMEM((MAXR,), jnp.float32),
        pltpu.VMEM((GCH, D), jnp.float32),
        pltpu.SemaphoreType.DMA,
    ],
    compiler_params=pltpu.CompilerParams(needs_layout_passes=False),
  )

  @kern
  def _dispatch(x_hbm, p0_hbm, p1_hbm, w0_hbm, w1_hbm, xs_hbm, ws_hbm,
                p0_v, p1_v, w0_v, w1_v, src_v, wso_v, gbuf, sem):
    wid = lax.axis_index("s") * 2 + lax.axis_index("c")
    pltpu.sync_copy(p0_hbm, p0_v)
    pltpu.sync_copy(p1_hbm, p1_v)
    pltpu.sync_copy(w0_hbm, w0_v)
    pltpu.sync_copy(w1_hbm, w1_v)

    zi = jnp.zeros((16,), jnp.int32)
    zf = jnp.zeros((16,), jnp.float32)

    def initb(i, _):
        src_v[pl.ds(i * 16, 16)] = zi
        wso_v[pl.ds(i * 16, 16)] = zf
        return 0

    lax.fori_loop(0, MAXR // 16, initb, 0)
    lane = lax.iota(jnp.int32, 16)

    def scat(i, _):
        t = lane + i * 16
        idx0 = p0_v[pl.ds(i * 16, 16)]
        plsc.store_scatter(src_v, [idx0], t)
        plsc.store_scatter(wso_v, [idx0], w0_v[pl.ds(i * 16, 16)])
        idx1 = p1_v[pl.ds(i * 16, 16)]
        plsc.store_scatter(src_v, [idx1], t)
        plsc.store_scatter(wso_v, [idx1], w1_v[pl.ds(i * 16, 16)])
        return 0

    lax.fori_loop(0, N_TOK // 16, scat, 0)
    pltpu.sync_copy(wso_v.at[pl.ds(wid * RPW, RPW)],
                    ws_hbm.at[pl.ds(wid * RPW, RPW)])

    def gchunk(k, _):
        r0 = wid * RPW + k * GCH
        pltpu.async_copy(x_hbm.at[src_v.at[pl.ds(r0, GCH)]], gbuf, sem).wait()
        pltpu.sync_copy(gbuf, xs_hbm.at[pl.ds(r0, GCH)])
        return 0

    lax.fori_loop(0, RPW // GCH, gchunk, 0)

  return _dispatch


def _ffn_body(emap_ref, used_ref, xs_ref, w1_ref, w3_ref, w2_ref, ws_ref,
              out_ref):
    b = pl.program_id(0)
    h = pl.program_id(1)

    @pl.when(b < used_ref[0])
    def _():
        xb = xs_ref[...]
        a = lax.dot_general(xb, w1_ref[0], (((1,), (1,)), ((), ())),
                            preferred_element_type=jnp.float32)
        c3 = lax.dot_general(xb, w3_ref[0], (((1,), (1,)), ((), ())),
                             preferred_element_type=jnp.float32)
        g = (a * jax.nn.sigmoid(a)) * c3 * ws_ref[...]
        o = lax.dot_general(g, w2_ref[0], (((1,), (1,)), ((), ())),
                            preferred_element_type=jnp.float32)

        @pl.when(h == 0)
        def _():
            out_ref[...] = o

        @pl.when(h > 0)
        def _():
            out_ref[...] = out_ref[...] + o


def _ffn(emap_arr, used_arr, xs, W1, W3, W2, ws):
    grid_spec = pltpu.PrefetchScalarGridSpec(
        num_scalar_prefetch=2,
        grid=(MAXB, NH),
        in_specs=[
            pl.BlockSpec((BM, D),
                         lambda b, h, em, us: (jnp.minimum(b, us[0] - 1), 0)),
            pl.BlockSpec((1, HB, D), lambda b, h, em, us: (em[b], h, 0)),
            pl.BlockSpec((1, HB, D), lambda b, h, em, us: (em[b], h, 0)),
            pl.BlockSpec((1, D, HB), lambda b, h, em, us: (em[b], 0, h)),
            pl.BlockSpec((BM, 1),
                         lambda b, h, em, us: (jnp.minimum(b, us[0] - 1), 0)),
        ],
        out_specs=pl.BlockSpec(
            (BM, D), lambda b, h, em, us: (jnp.minimum(b, us[0] - 1), 0)),
    )
    return pl.pallas_call(
        _ffn_body,
        grid_spec=grid_spec,
        out_shape=jax.ShapeDtypeStruct((MAXR, D), jnp.float32),
        compiler_params=pltpu.CompilerParams(
            dimension_semantics=("arbitrary", "arbitrary")),
    )(emap_arr, used_arr, xs, W1, W3, W2, ws)


@functools.cache
def _make_combine():
  kern = functools.partial(
    pl.kernel,
    mesh=plsc.VectorSubcoreMesh(**_SC_MESH),
    out_type=jax.ShapeDtypeStruct((N_TOK, D), jnp.float32),
    scratch_types=[
        pltpu.VMEM((TPW,), jnp.int32),
        pltpu.VMEM((TPW,), jnp.int32),
        pltpu.VMEM((TPW, D), jnp.float32),
        pltpu.VMEM((TPW, D), jnp.float32),
        pltpu.SemaphoreType.DMA,
        pltpu.SemaphoreType.DMA,
    ],
    compiler_params=pltpu.CompilerParams(needs_layout_passes=False),
  )

  @kern
  def _combine(ys_hbm, p0_hbm, p1_hbm, out_hbm, i0_v, i1_v, b0, b1, s0, s1):
    wid = lax.axis_index("s") * 2 + lax.axis_index("c")
    t0 = wid * TPW
    pltpu.sync_copy(p0_hbm.at[pl.ds(t0, TPW)], i0_v)
    pltpu.sync_copy(p1_hbm.at[pl.ds(t0, TPW)], i1_v)
    cp0 = pltpu.async_copy(ys_hbm.at[i0_v], b0, s0)
    cp1 = pltpu.async_copy(ys_hbm.at[i1_v], b1, s1)
    cp0.wait()
    cp1.wait()

    def row(r, _):
        def col(c, _2):
            sl = pl.ds(c * 16, 16)
            b0[r, sl] = b0[r, sl] + b1[r, sl]
            return 0

        lax.fori_loop(0, D // 16, col, 0)
        return 0

    lax.fori_loop(0, TPW, row, 0)
    pltpu.sync_copy(b0, out_hbm.at[pl.ds(t0, TPW)])

  return _combine


def kernel(x, Wg, W1, W2, W3):
    B, T, C = x.shape
    x2d = x.reshape(T, C)
    pos0, pos1, w0, w1, meta = _route(x2d, Wg)
    p0 = pos0.reshape(T)
    p1 = pos1.reshape(T)
    emap_arr = meta[0, :32]
    used_arr = meta[0, 32:33]
    xs, ws = _make_dispatch()(x2d, p0, p1, w0.reshape(T), w1.reshape(T))
    ys = _ffn(emap_arr, used_arr, xs, W1, W3, W2, ws.reshape(MAXR, 1))
    out = _make_combine()(ys, p0, p1)
    return out.reshape(B, T, C)


# speedup vs baseline: 1.3014x; 1.3014x over previous
import functools

import jax
import jax.numpy as jnp
from jax import lax
from jax.experimental import pallas as pl
from jax.experimental.pallas import tpu as pltpu
from jax.experimental.pallas import tpu_sc as plsc

N_TOK = 2048
D = 768
E = 8
H = 2048
K = 2
A_TOT = N_TOK * K
BM = 256
MAXB = A_TOT // BM + E
MAXR = MAXB * BM
HB = 2048
NH = H // HB

NW = 32
RPW = MAXR // NW
TPW = N_TOK // NW
GCH = 96

_SC_MESH = dict(core_axis_name="c", subcore_axis_name="s", num_cores=2,
                num_subcores=16)


def _route_body(x_ref, wg_ref, pos0_ref, pos1_ref, w0_ref, w1_ref, meta_ref,
                cum_ref):
    xl = x_ref[...]
    wg = wg_ref[...]
    logits = lax.dot_general(xl, wg, (((1,), (1,)), ((), ())),
                             preferred_element_type=jnp.float32)
    e_io = lax.broadcasted_iota(jnp.int32, (N_TOK, E), 1)
    m0 = jnp.max(logits, axis=1, keepdims=True)
    i0 = jnp.min(jnp.where(logits == m0, e_io, E), axis=1, keepdims=True)
    masked = jnp.where(e_io == i0, -jnp.inf, logits)
    m1 = jnp.max(masked, axis=1, keepdims=True)
    i1 = jnp.min(jnp.where(masked == m1, e_io, E), axis=1, keepdims=True)
    a = jnp.exp(m1 - m0)
    w0_ref[...] = 1.0 / (1.0 + a)
    w1_ref[...] = a / (1.0 + a)

    oh0 = (e_io == i0).astype(jnp.float32)
    oh1 = (e_io == i1).astype(jnp.float32)
    oh = jnp.concatenate([oh0, oh1], axis=1)
    tri = (lax.broadcasted_iota(jnp.int32, (128, 128), 0)
           >= lax.broadcasted_iota(jnp.int32, (128, 128), 1)
           ).astype(jnp.float32)

    cum_ref[...] = oh

    def chunk(c, run):
        blk = cum_ref[pl.ds(c * 128, 128), :]
        p = lax.dot_general(tri, blk, (((1,), (0,)), ((), ())),
                            preferred_element_type=jnp.float32) + run
        cum_ref[pl.ds(c * 128, 128), :] = p
        return p[127:128, :]

    run = lax.fori_loop(0, N_TOK // 128, chunk, jnp.zeros((1, 2 * E),
                                                          jnp.float32))
    cum = cum_ref[...]
    cum0, cum1 = cum[:, 0:E], cum[:, E:2 * E]
    counts0, counts1 = run[:, 0:E], run[:, E:2 * E]
    counts = counts0 + counts1
    nblk = jnp.floor((counts + (BM - 1)) * (1.0 / BM))
    ut = (lax.broadcasted_iota(jnp.int32, (E, E), 0)
          <= lax.broadcasted_iota(jnp.int32, (E, E), 1)).astype(jnp.float32)
    cumblk = lax.dot_general(nblk, ut, (((1,), (0,)), ((), ())),
                             preferred_element_type=jnp.float32)
    pad_off = (cumblk - nblk) * BM
    used = cumblk[:, E - 1:E]

    base0 = pad_off + cum0 - 1.0
    base1 = pad_off + counts0 + cum1 - 1.0
    pos0_ref[...] = jnp.sum(oh0 * base0, axis=1, keepdims=True).astype(
        jnp.int32)
    pos1_ref[...] = jnp.sum(oh1 * base1, axis=1, keepdims=True).astype(
        jnp.int32)

    eye8 = (lax.broadcasted_iota(jnp.int32, (E, E), 0)
            == lax.broadcasted_iota(jnp.int32, (E, E), 1)).astype(jnp.float32)
    cumblk_col = lax.dot_general(eye8, cumblk, (((1,), (1,)), ((), ())),
                                 preferred_element_type=jnp.float32)
    counts_col = lax.dot_general(eye8, counts, (((1,), (1,)), ((), ())),
                                 preferred_element_type=jnp.float32)
    b_io = lax.broadcasted_iota(jnp.int32, (E, 32), 1).astype(jnp.float32)
    ebm = jnp.sum((b_io >= cumblk_col).astype(jnp.float32), axis=0,
                  keepdims=True)
    e_col = lax.broadcasted_iota(jnp.int32, (E, 1), 0).astype(jnp.float32)
    last_e = jnp.max(jnp.where(counts_col > 0.0, e_col, 0.0), axis=0,
                     keepdims=True)
    b_row = lax.broadcasted_iota(jnp.int32, (1, 32), 1).astype(jnp.float32)
    emap = jnp.where(b_row >= used, last_e, ebm)
    meta_ref[...] = jnp.concatenate(
        [emap, jnp.broadcast_to(used, (1, 32))], axis=1).astype(jnp.int32)


def _route(x2d, Wg):
    return pl.pallas_call(
        _route_body,
        out_shape=[
            jax.ShapeDtypeStruct((N_TOK, 1), jnp.int32),
            jax.ShapeDtypeStruct((N_TOK, 1), jnp.int32),
            jax.ShapeDtypeStruct((N_TOK, 1), jnp.float32),
            jax.ShapeDtypeStruct((N_TOK, 1), jnp.float32),
            jax.ShapeDtypeStruct((1, 64), jnp.int32),
        ],
        scratch_shapes=[pltpu.VMEM((N_TOK, 2 * E), jnp.float32)],
    )(x2d, Wg)


GB = 64


@functools.cache
def _make_dispatch():
  kern = functools.partial(
    pl.kernel,
    mesh=plsc.VectorSubcoreMesh(**_SC_MESH),
    out_type=[
        jax.ShapeDtypeStruct((MAXR, D), jnp.float32),
        jax.ShapeDtypeStruct((MAXR,), jnp.float32),
    ],
    scratch_types=[
        pltpu.VMEM((128,), jnp.int32),
        pltpu.VMEM((128,), jnp.int32),
        pltpu.VMEM((128,), jnp.float32),
        pltpu.VMEM((128,), jnp.float32),
        pltpu.VMEM((128,), jnp.int32),
        pltpu.VMEM((MAXR // 16,), jnp.int32),
        pltpu.VMEM((MAXR // 16,), jnp.float32),
        pltpu.VMEM((RPW,), jnp.int32),
        pltpu.VMEM((RPW,), jnp.float32),
        pltpu.VMEM((GB, D), jnp.float32),
        pltpu.VMEM((GB, D), jnp.float32),
        pltpu.VMEM_SHARED((MAXR,), jnp.int32),
        pltpu.VMEM_SHARED((MAXR,), jnp.float32),
        pltpu.SemaphoreType.DMA,
        pltpu.SemaphoreType.DMA,
        pltpu.SemaphoreType.DMA,
        pltpu.SemaphoreType.DMA,
    ],
    compiler_params=pltpu.CompilerParams(needs_layout_passes=False),
  )

  @kern
  def _dispatch(x_hbm, p0_hbm, p1_hbm, w0_hbm, w1_hbm, xs_hbm, ws_hbm,
                i0_v, i1_v, w0_v, w1_v, tid_v, zbi, zbf, srcl, wsl,
                gb0, gb1, src_sh, wso_sh, sem0, sem1, semw0, semw1):
    cid = lax.axis_index("c")
    sid = lax.axis_index("s")
    wid = sid * 2 + cid
    tok0 = sid * 128
    with jax.named_scope("b_load"):
      pltpu.sync_copy(p0_hbm.at[pl.ds(tok0, 128)], i0_v)
      pltpu.sync_copy(p1_hbm.at[pl.ds(tok0, 128)], i1_v)
      pltpu.sync_copy(w0_hbm.at[pl.ds(tok0, 128)], w0_v)
      pltpu.sync_copy(w1_hbm.at[pl.ds(tok0, 128)], w1_v)

    zi = jnp.zeros((16,), jnp.int32)
    zf = jnp.zeros((16,), jnp.float32)
    lane = lax.iota(jnp.int32, 16)

    def initb(i, _):
        zbi[pl.ds(i * 16, 16)] = zi
        zbf[pl.ds(i * 16, 16)] = zf
        return 0

    lax.fori_loop(0, MAXR // 256, initb, 0)

    def tids(i, _):
        tid_v[pl.ds(i * 16, 16)] = lane + (tok0 + i * 16)
        return 0

    lax.fori_loop(0, 8, tids, 0)

    with jax.named_scope("b_scatter"):
      z0 = sid * (MAXR // 16)
      pltpu.sync_copy(zbi, src_sh.at[pl.ds(z0, MAXR // 16)])
      pltpu.sync_copy(zbf, wso_sh.at[pl.ds(z0, MAXR // 16)])
      plsc.subcore_barrier()
      pltpu.sync_copy(tid_v, src_sh.at[i0_v])
      pltpu.sync_copy(tid_v, src_sh.at[i1_v])
      pltpu.sync_copy(w0_v, wso_sh.at[i0_v])
      pltpu.sync_copy(w1_v, wso_sh.at[i1_v])
      plsc.subcore_barrier()

    with jax.named_scope("b_readback"):
      r0 = wid * RPW
      pltpu.sync_copy(src_sh.at[pl.ds(r0, RPW)], srcl)
      pltpu.sync_copy(wso_sh.at[pl.ds(r0, RPW)], wsl)
      pltpu.sync_copy(wsl, ws_hbm.at[pl.ds(r0, RPW)])

    with jax.named_scope("b_gather"):
      cp0 = pltpu.async_copy(x_hbm.at[srcl.at[pl.ds(0, GB)]], gb0, sem0)
      cp1 = pltpu.async_copy(x_hbm.at[srcl.at[pl.ds(GB, GB)]], gb1, sem1)
      cp0.wait()
      wr0 = pltpu.async_copy(gb0, xs_hbm.at[pl.ds(r0, GB)], semw0)
      cp1.wait()
      wr1 = pltpu.async_copy(gb1, xs_hbm.at[pl.ds(r0 + GB, GB)], semw1)
      wr0.wait()
      cp2 = pltpu.async_copy(x_hbm.at[srcl.at[pl.ds(2 * GB, GB)]], gb0, sem0)
      cp2.wait()
      wr2 = pltpu.async_copy(gb0, xs_hbm.at[pl.ds(r0 + 2 * GB, GB)], semw0)
      wr1.wait()
      wr2.wait()

  return _dispatch


def _ffn_body(emap_ref, used_ref, xs_ref, w1_ref, w3_ref, w2_ref, ws_ref,
              out_ref):
    b = pl.program_id(0)

    @pl.when(b < used_ref[0])
    def _():
        xb = xs_ref[...]
        a = lax.dot_general(xb, w1_ref[0], (((1,), (1,)), ((), ())),
                            preferred_element_type=jnp.float32)
        c3 = lax.dot_general(xb, w3_ref[0], (((1,), (1,)), ((), ())),
                             preferred_element_type=jnp.float32)
        g = (a * jax.nn.sigmoid(a)) * c3 * ws_ref[...]
        out_ref[...] = lax.dot_general(g, w2_ref[0], (((1,), (1,)), ((), ())),
                                       preferred_element_type=jnp.float32)


def _ffn(emap_arr, used_arr, xs, W1, W3, W2, ws):
    grid_spec = pltpu.PrefetchScalarGridSpec(
        num_scalar_prefetch=2,
        grid=(MAXB,),
        in_specs=[
            pl.BlockSpec((BM, D), lambda b, em, us: (jnp.minimum(b, us[0] - 1), 0)),
            pl.BlockSpec((1, H, D), lambda b, em, us: (em[b], 0, 0)),
            pl.BlockSpec((1, H, D), lambda b, em, us: (em[b], 0, 0)),
            pl.BlockSpec((1, D, H), lambda b, em, us: (em[b], 0, 0)),
            pl.BlockSpec((BM, 1), lambda b, em, us: (jnp.minimum(b, us[0] - 1), 0)),
        ],
        out_specs=pl.BlockSpec(
            (BM, D), lambda b, em, us: (jnp.minimum(b, us[0] - 1), 0)),
    )
    return pl.pallas_call(
        _ffn_body,
        grid_spec=grid_spec,
        out_shape=jax.ShapeDtypeStruct((MAXR, D), jnp.float32),
        compiler_params=pltpu.CompilerParams(
            dimension_semantics=("arbitrary",)),
    )(emap_arr, used_arr, xs, W1, W3, W2, ws)


@functools.cache
def _make_combine():
  kern = functools.partial(
    pl.kernel,
    mesh=plsc.VectorSubcoreMesh(**_SC_MESH),
    out_type=jax.ShapeDtypeStruct((N_TOK, D), jnp.float32),
    scratch_types=[
        pltpu.VMEM((TPW,), jnp.int32),
        pltpu.VMEM((TPW,), jnp.int32),
        pltpu.VMEM((TPW, D), jnp.float32),
        pltpu.VMEM((TPW, D), jnp.float32),
        pltpu.SemaphoreType.DMA,
        pltpu.SemaphoreType.DMA,
    ],
    compiler_params=pltpu.CompilerParams(needs_layout_passes=False),
  )

  @kern
  def _combine(ys_hbm, p0_hbm, p1_hbm, out_hbm, i0_v, i1_v, b0, b1, s0, s1):
    wid = lax.axis_index("s") * 2 + lax.axis_index("c")
    t0 = wid * TPW
    pltpu.sync_copy(p0_hbm.at[pl.ds(t0, TPW)], i0_v)
    pltpu.sync_copy(p1_hbm.at[pl.ds(t0, TPW)], i1_v)
    cp0 = pltpu.async_copy(ys_hbm.at[i0_v], b0, s0)
    cp1 = pltpu.async_copy(ys_hbm.at[i1_v], b1, s1)
    cp0.wait()
    cp1.wait()

    def row(r, _):
        def col(c, _2):
            sl = pl.ds(c * 16, 16)
            b0[r, sl] = b0[r, sl] + b1[r, sl]
            return 0

        lax.fori_loop(0, D // 16, col, 0)
        return 0

    lax.fori_loop(0, TPW, row, 0)
    pltpu.sync_copy(b0, out_hbm.at[pl.ds(t0, TPW)])

  return _combine


def kernel(x, Wg, W1, W2, W3):
    B, T, C = x.shape
    x2d = x.reshape(T, C)
    pos0, pos1, w0, w1, meta = _route(x2d, Wg)
    p0 = pos0.reshape(T)
    p1 = pos1.reshape(T)
    emap_arr = meta[0, :32]
    used_arr = meta[0, 32:33]
    xs, ws = _make_dispatch()(x2d, p0, p1, w0.reshape(T), w1.reshape(T))
    ys = _ffn(emap_arr, used_arr, xs, W1, W3, W2, ws.reshape(MAXR, 1))
    out = _make_combine()(ys, p0, p1)
    return out.reshape(B, T, C)


# speedup vs baseline: 1.3071x; 1.0043x over previous
import functools

import jax
import jax.numpy as jnp
from jax import lax
from jax.experimental import pallas as pl
from jax.experimental.pallas import tpu as pltpu
from jax.experimental.pallas import tpu_sc as plsc

N_TOK = 2048
D = 768
E = 8
H = 2048
K = 2
A_TOT = N_TOK * K
BM = 256
MAXB = A_TOT // BM + E
MAXR = MAXB * BM
HB = 2048
NH = H // HB

NW = 32
RPW = MAXR // NW
TPW = N_TOK // NW
GCH = 96

_SC_MESH = dict(core_axis_name="c", subcore_axis_name="s", num_cores=2,
                num_subcores=16)


def _route_body(x_ref, wg_ref, pos0_ref, pos1_ref, w0_ref, w1_ref, meta_ref,
                cum_ref):
    xl = x_ref[...]
    wg = wg_ref[...]
    logits = lax.dot_general(xl, wg, (((1,), (1,)), ((), ())),
                             preferred_element_type=jnp.float32)
    e_io = lax.broadcasted_iota(jnp.int32, (N_TOK, E), 1)
    m0 = jnp.max(logits, axis=1, keepdims=True)
    i0 = jnp.min(jnp.where(logits == m0, e_io, E), axis=1, keepdims=True)
    masked = jnp.where(e_io == i0, -jnp.inf, logits)
    m1 = jnp.max(masked, axis=1, keepdims=True)
    i1 = jnp.min(jnp.where(masked == m1, e_io, E), axis=1, keepdims=True)
    a = jnp.exp(m1 - m0)
    w0_ref[...] = 1.0 / (1.0 + a)
    w1_ref[...] = a / (1.0 + a)

    oh0 = (e_io == i0).astype(jnp.float32)
    oh1 = (e_io == i1).astype(jnp.float32)
    oh = jnp.concatenate([oh0, oh1], axis=1)
    tri = (lax.broadcasted_iota(jnp.int32, (128, 128), 0)
           >= lax.broadcasted_iota(jnp.int32, (128, 128), 1)
           ).astype(jnp.float32)

    cum_ref[...] = oh

    def chunk(c, run):
        blk = cum_ref[pl.ds(c * 128, 128), :]
        p = lax.dot_general(tri, blk, (((1,), (0,)), ((), ())),
                            preferred_element_type=jnp.float32) + run
        cum_ref[pl.ds(c * 128, 128), :] = p
        return p[127:128, :]

    run = lax.fori_loop(0, N_TOK // 128, chunk, jnp.zeros((1, 2 * E),
                                                          jnp.float32))
    cum = cum_ref[...]
    cum0, cum1 = cum[:, 0:E], cum[:, E:2 * E]
    counts0, counts1 = run[:, 0:E], run[:, E:2 * E]
    counts = counts0 + counts1
    nblk = jnp.floor((counts + (BM - 1)) * (1.0 / BM))
    ut = (lax.broadcasted_iota(jnp.int32, (E, E), 0)
          <= lax.broadcasted_iota(jnp.int32, (E, E), 1)).astype(jnp.float32)
    cumblk = lax.dot_general(nblk, ut, (((1,), (0,)), ((), ())),
                             preferred_element_type=jnp.float32)
    pad_off = (cumblk - nblk) * BM
    used = cumblk[:, E - 1:E]

    base0 = pad_off + cum0 - 1.0
    base1 = pad_off + counts0 + cum1 - 1.0
    pos0_ref[...] = jnp.sum(oh0 * base0, axis=1, keepdims=True).astype(
        jnp.int32)
    pos1_ref[...] = jnp.sum(oh1 * base1, axis=1, keepdims=True).astype(
        jnp.int32)

    eye8 = (lax.broadcasted_iota(jnp.int32, (E, E), 0)
            == lax.broadcasted_iota(jnp.int32, (E, E), 1)).astype(jnp.float32)
    cumblk_col = lax.dot_general(eye8, cumblk, (((1,), (1,)), ((), ())),
                                 preferred_element_type=jnp.float32)
    counts_col = lax.dot_general(eye8, counts, (((1,), (1,)), ((), ())),
                                 preferred_element_type=jnp.float32)
    b_io = lax.broadcasted_iota(jnp.int32, (E, 32), 1).astype(jnp.float32)
    ebm = jnp.sum((b_io >= cumblk_col).astype(jnp.float32), axis=0,
                  keepdims=True)
    e_col = lax.broadcasted_iota(jnp.int32, (E, 1), 0).astype(jnp.float32)
    last_e = jnp.max(jnp.where(counts_col > 0.0, e_col, 0.0), axis=0,
                     keepdims=True)
    b_row = lax.broadcasted_iota(jnp.int32, (1, 32), 1).astype(jnp.float32)
    emap = jnp.where(b_row >= used, last_e, ebm)
    meta_ref[...] = jnp.concatenate(
        [emap, jnp.broadcast_to(used, (1, 32))], axis=1).astype(jnp.int32)


def _route(x2d, Wg):
    return pl.pallas_call(
        _route_body,
        out_shape=[
            jax.ShapeDtypeStruct((N_TOK, 1), jnp.int32),
            jax.ShapeDtypeStruct((N_TOK, 1), jnp.int32),
            jax.ShapeDtypeStruct((N_TOK, 1), jnp.float32),
            jax.ShapeDtypeStruct((N_TOK, 1), jnp.float32),
            jax.ShapeDtypeStruct((1, 64), jnp.int32),
        ],
        scratch_shapes=[pltpu.VMEM((N_TOK, 2 * E), jnp.float32)],
    )(x2d, Wg)


GB = 48


@functools.cache
def _make_dispatch():
  kern = functools.partial(
    pl.kernel,
    mesh=plsc.VectorSubcoreMesh(**_SC_MESH),
    out_type=[
        jax.ShapeDtypeStruct((MAXR, D), jnp.float32),
        jax.ShapeDtypeStruct((MAXR,), jnp.float32),
    ],
    scratch_types=[
        pltpu.VMEM((128,), jnp.int32),
        pltpu.VMEM((128,), jnp.int32),
        pltpu.VMEM((128,), jnp.float32),
        pltpu.VMEM((128,), jnp.float32),
        pltpu.VMEM((128,), jnp.int32),
        pltpu.VMEM((MAXR // 16,), jnp.int32),
        pltpu.VMEM((MAXR // 16,), jnp.float32),
        pltpu.VMEM((RPW,), jnp.int32),
        pltpu.VMEM((RPW,), jnp.float32),
        pltpu.VMEM((GB, D), jnp.float32),
        pltpu.VMEM((GB, D), jnp.float32),
        pltpu.VMEM_SHARED((MAXR,), jnp.int32),
        pltpu.VMEM_SHARED((MAXR,), jnp.float32),
        pltpu.SemaphoreType.DMA,
        pltpu.SemaphoreType.DMA,
        pltpu.SemaphoreType.DMA,
        pltpu.SemaphoreType.DMA,
    ],
    compiler_params=pltpu.CompilerParams(needs_layout_passes=False),
  )

  @kern
  def _dispatch(x_hbm, p0_hbm, p1_hbm, w0_hbm, w1_hbm, xs_hbm, ws_hbm,
                i0_v, i1_v, w0_v, w1_v, tid_v, zbi, zbf, srcl, wsl,
                gb0, gb1, src_sh, wso_sh, sem0, sem1, semw0, semw1):
    cid = lax.axis_index("c")
    sid = lax.axis_index("s")
    wid = sid * 2 + cid
    tok0 = sid * 128
    with jax.named_scope("b_load"):
      pltpu.sync_copy(p0_hbm.at[pl.ds(tok0, 128)], i0_v)
      pltpu.sync_copy(p1_hbm.at[pl.ds(tok0, 128)], i1_v)
      pltpu.sync_copy(w0_hbm.at[pl.ds(tok0, 128)], w0_v)
      pltpu.sync_copy(w1_hbm.at[pl.ds(tok0, 128)], w1_v)

    zi = jnp.zeros((16,), jnp.int32)
    zf = jnp.zeros((16,), jnp.float32)
    lane = lax.iota(jnp.int32, 16)

    def initb(i, _):
        zbi[pl.ds(i * 16, 16)] = zi
        zbf[pl.ds(i * 16, 16)] = zf
        return 0

    lax.fori_loop(0, MAXR // 256, initb, 0)

    def tids(i, _):
        tid_v[pl.ds(i * 16, 16)] = lane + (tok0 + i * 16)
        return 0

    lax.fori_loop(0, 8, tids, 0)

    with jax.named_scope("b_scatter"):
      z0 = sid * (MAXR // 16)
      pltpu.sync_copy(zbi, src_sh.at[pl.ds(z0, MAXR // 16)])
      pltpu.sync_copy(zbf, wso_sh.at[pl.ds(z0, MAXR // 16)])
      plsc.subcore_barrier()
      pltpu.sync_copy(tid_v, src_sh.at[i0_v])
      pltpu.sync_copy(tid_v, src_sh.at[i1_v])
      pltpu.sync_copy(w0_v, wso_sh.at[i0_v])
      pltpu.sync_copy(w1_v, wso_sh.at[i1_v])
      plsc.subcore_barrier()

    with jax.named_scope("b_readback"):
      r0 = wid * RPW
      pltpu.sync_copy(src_sh.at[pl.ds(r0, RPW)], srcl)
      pltpu.sync_copy(wso_sh.at[pl.ds(r0, RPW)], wsl)
      pltpu.sync_copy(wsl, ws_hbm.at[pl.ds(r0, RPW)])

    with jax.named_scope("b_gather"):
      cp0 = pltpu.async_copy(x_hbm.at[srcl.at[pl.ds(0, GB)]], gb0, sem0)
      cp1 = pltpu.async_copy(x_hbm.at[srcl.at[pl.ds(GB, GB)]], gb1, sem1)
      cp0.wait()
      wr0 = pltpu.async_copy(gb0, xs_hbm.at[pl.ds(r0, GB)], semw0)
      cp1.wait()
      wr1 = pltpu.async_copy(gb1, xs_hbm.at[pl.ds(r0 + GB, GB)], semw1)
      wr0.wait()
      cp2 = pltpu.async_copy(x_hbm.at[srcl.at[pl.ds(2 * GB, GB)]], gb0, sem0)
      wr1.wait()
      cp3 = pltpu.async_copy(x_hbm.at[srcl.at[pl.ds(3 * GB, GB)]], gb1, sem1)
      cp2.wait()
      wr2 = pltpu.async_copy(gb0, xs_hbm.at[pl.ds(r0 + 2 * GB, GB)], semw0)
      cp3.wait()
      wr3 = pltpu.async_copy(gb1, xs_hbm.at[pl.ds(r0 + 3 * GB, GB)], semw1)
      wr2.wait()
      wr3.wait()

  return _dispatch


def _ffn_body(emap_ref, used_ref, xs_ref, w1_ref, w3_ref, w2_ref, ws_ref,
              out_ref):
    b = pl.program_id(0)

    @pl.when(b < used_ref[0])
    def _():
        xb = xs_ref[...]
        a = lax.dot_general(xb, w1_ref[0], (((1,), (1,)), ((), ())),
                            preferred_element_type=jnp.float32)
        c3 = lax.dot_general(xb, w3_ref[0], (((1,), (1,)), ((), ())),
                             preferred_element_type=jnp.float32)
        g = (a * jax.nn.sigmoid(a)) * c3 * ws_ref[...]
        out_ref[...] = lax.dot_general(g, w2_ref[0], (((1,), (1,)), ((), ())),
                                       preferred_element_type=jnp.float32)


def _ffn(emap_arr, used_arr, xs, W1, W3, W2, ws):
    grid_spec = pltpu.PrefetchScalarGridSpec(
        num_scalar_prefetch=2,
        grid=(MAXB,),
        in_specs=[
            pl.BlockSpec((BM, D), lambda b, em, us: (jnp.minimum(b, us[0] - 1), 0)),
            pl.BlockSpec((1, H, D), lambda b, em, us: (em[b], 0, 0)),
            pl.BlockSpec((1, H, D), lambda b, em, us: (em[b], 0, 0)),
            pl.BlockSpec((1, D, H), lambda b, em, us: (em[b], 0, 0)),
            pl.BlockSpec((BM, 1), lambda b, em, us: (jnp.minimum(b, us[0] - 1), 0)),
        ],
        out_specs=pl.BlockSpec(
            (BM, D), lambda b, em, us: (jnp.minimum(b, us[0] - 1), 0)),
    )
    return pl.pallas_call(
        _ffn_body,
        grid_spec=grid_spec,
        out_shape=jax.ShapeDtypeStruct((MAXR, D), jnp.float32),
        compiler_params=pltpu.CompilerParams(
            dimension_semantics=("arbitrary",)),
    )(emap_arr, used_arr, xs, W1, W3, W2, ws)


@functools.cache
def _make_combine():
  kern = functools.partial(
    pl.kernel,
    mesh=plsc.VectorSubcoreMesh(**_SC_MESH),
    out_type=jax.ShapeDtypeStruct((N_TOK, D), jnp.float32),
    scratch_types=[
        pltpu.VMEM((TPW,), jnp.int32),
        pltpu.VMEM((TPW,), jnp.int32),
        pltpu.VMEM((TPW, D), jnp.float32),
        pltpu.VMEM((TPW, D), jnp.float32),
        pltpu.SemaphoreType.DMA,
        pltpu.SemaphoreType.DMA,
    ],
    compiler_params=pltpu.CompilerParams(needs_layout_passes=False),
  )

  @kern
  def _combine(ys_hbm, p0_hbm, p1_hbm, out_hbm, i0_v, i1_v, b0, b1, s0, s1):
    wid = lax.axis_index("s") * 2 + lax.axis_index("c")
    t0 = wid * TPW
    pltpu.sync_copy(p0_hbm.at[pl.ds(t0, TPW)], i0_v)
    pltpu.sync_copy(p1_hbm.at[pl.ds(t0, TPW)], i1_v)
    cp0 = pltpu.async_copy(ys_hbm.at[i0_v], b0, s0)
    cp1 = pltpu.async_copy(ys_hbm.at[i1_v], b1, s1)
    cp0.wait()
    cp1.wait()

    def row(r, _):
        def col(c, _2):
            sl = pl.ds(c * 16, 16)
            b0[r, sl] = b0[r, sl] + b1[r, sl]
            return 0

        lax.fori_loop(0, D // 16, col, 0)
        return 0

    lax.fori_loop(0, TPW, row, 0)
    pltpu.sync_copy(b0, out_hbm.at[pl.ds(t0, TPW)])

  return _combine


def kernel(x, Wg, W1, W2, W3):
    B, T, C = x.shape
    x2d = x.reshape(T, C)
    pos0, pos1, w0, w1, meta = _route(x2d, Wg)
    p0 = pos0.reshape(T)
    p1 = pos1.reshape(T)
    emap_arr = meta[0, :32]
    used_arr = meta[0, 32:33]
    xs, ws = _make_dispatch()(x2d, p0, p1, w0.reshape(T), w1.reshape(T))
    ys = _ffn(emap_arr, used_arr, xs, W1, W3, W2, ws.reshape(MAXR, 1))
    out = _make_combine()(ys, p0, p1)
    return out.reshape(B, T, C)
